# baseline (speedup 1.0000x reference)
import functools

import jax
import jax.numpy as jnp
from jax import lax
from jax.experimental import pallas as pl
from jax.experimental.pallas import tpu as pltpu
from jax.experimental.pallas import tpu_sc as plsc

_BATCH = 64
_THRESH2 = 3.0
_ALPHA = 2.0
_BETA = 1.0
_OBS_R = 256
_OBS_S = 512

_NS = 16
_NT = _BATCH - _NS
_W = 32 // _NS
_RPI = _OBS_R // _W
_CR = 32
_NCH = _RPI // _CR
_UNROLL = 8
_GRP = _OBS_S // 16


def _tc_score(z_ref, r_ref, out_ref):
    x = z_ref[0]
    x0 = x[0]
    x1 = x[1]
    x2 = x[2]
    z2 = x0 * x0 + x1 * x1 + x2 * x2
    b_coef = -0.5 / (r_ref[0, 0, 0] * r_ref[0, 0, 0])
    e = jnp.exp(z2 * b_coef)
    out_ref[0, 0, 0] = jnp.sum(jnp.where(z2 < _THRESH2, e, 0.0))


def _sc_score(z_hbm, bc_hbm, out_hbm, b0a, b0b, b1a, b1b, b2a, b2b,
              bc_v, o_v, sem_a, sem_b):
    wid = lax.axis_index("s") * 2 + lax.axis_index("c")
    b = _NT + wid // _W
    h = wid % _W
    r0 = h * _RPI
    bufs = [(b0a, b1a, b2a), (b0b, b1b, b2b)]
    sems = [sem_a, sem_b]

    pltpu.sync_copy(bc_hbm.at[b], bc_v)
    bcoef = bc_v[...]

    def start(c, d):
        rs = r0 + c * _CR
        return tuple(
            pltpu.async_copy(z_hbm.at[b, k, pl.ds(rs, _CR)],
                             bufs[d][k], sems[d])
            for k in range(3))

    handles = {0: start(0, 0)}
    acc = (jnp.zeros((16,), jnp.float32),) * 4
    for c in range(_NCH):
        d = c % 2
        if c + 1 < _NCH:
            handles[c + 1] = start(c + 1, (c + 1) % 2)
        for hd in handles.pop(c):
            hd.wait()
        b0, b1, b2 = bufs[d]

        def row_step(r, carry):
            def grp_step(g, carry2):
                accs = list(carry2)
                base = g * (16 * _UNROLL)
                for u in range(_UNROLL):
                    sl = pl.ds(base + 16 * u, 16)
                    x0 = b0[r, sl]
                    x1 = b1[r, sl]
                    x2 = b2[r, sl]
                    z2 = x0 * x0 + x1 * x1 + x2 * x2
                    e = jnp.exp(z2 * bcoef)
                    accs[u % 4] = accs[u % 4] + jnp.where(
                        z2 < _THRESH2, e, 0.0)
                return tuple(accs)

            return lax.fori_loop(0, _GRP // _UNROLL, grp_step, carry)

        acc = lax.fori_loop(0, _CR, row_step, acc)
    o_v[...] = (acc[0] + acc[1]) + (acc[2] + acc[3])
    pltpu.sync_copy(o_v, out_hbm.at[b - _NT, h])


def _finish_body(rawtc_ref, part_ref, r_ref, nobs_ref, raw_ref, mu_ref,
                 s2_ref, obj_ref):
    raw_sc = jnp.sum(part_ref[...], axis=1, keepdims=True)
    raw = jnp.concatenate([rawtc_ref[...], raw_sc], axis=0)
    r = r_ref[...]
    a = 1.0 / (r * r)
    b = 0.5 * a
    t2 = _THRESH2
    mu = (1.0 - jnp.exp(-b * t2)) / (4.0 * b)
    mean_s2 = (1.0 - jnp.exp(-2.0 * b * t2)) / (8.0 * b)
    sigma2 = mean_s2 - mu * mu
    n = nobs_ref[0, 0]
    mu = n * mu
    sigma2 = n * sigma2
    raw_ref[...] = raw
    mu_ref[...] = mu
    s2_ref[...] = sigma2
    obj_ref[...] = raw - _ALPHA * mu - _BETA * sigma2


@functools.partial(jax.jit, static_argnames=())
def kernel(z, R, num_obs):
    zt = jnp.transpose(z, (0, 3, 1, 2))
    bcast = jnp.broadcast_to((-0.5 / (R * R))[:, None], (_BATCH, 16))
    mesh = plsc.VectorSubcoreMesh(core_axis_name="c", subcore_axis_name="s")

    partials = pl.kernel(
        _sc_score,
        mesh=mesh,
        compiler_params=pltpu.CompilerParams(needs_layout_passes=False),
        out_type=jax.ShapeDtypeStruct((_NS, _W, 16), jnp.float32),
        scratch_types=(
            [pltpu.VMEM((_CR, _OBS_S), jnp.float32)] * 6
            + [pltpu.VMEM((16,), jnp.float32)] * 2
            + [pltpu.SemaphoreType.DMA] * 2
        ),
    )(zt, bcast)

    raw_tc = pl.pallas_call(
        _tc_score,
        grid=(_NT,),
        in_specs=[
            pl.BlockSpec((1, 3, _OBS_R, _OBS_S), lambda bb: (bb, 0, 0, 0)),
            pl.BlockSpec((1, 1, 1), lambda bb: (bb, 0, 0),
                         memory_space=pltpu.SMEM),
        ],
        out_specs=pl.BlockSpec((1, 1, 1), lambda bb: (bb, 0, 0),
                               memory_space=pltpu.SMEM),
        out_shape=jax.ShapeDtypeStruct((_NT, 1, 1), jnp.float32),
    )(zt, R.reshape(_BATCH, 1, 1))

    nobs = jnp.asarray(num_obs, jnp.float32).reshape(1, 1)
    raw, mu, sigma2, obj = pl.pallas_call(
        _finish_body,
        in_specs=[
            pl.BlockSpec((_NT, 1), lambda: (0, 0)),
            pl.BlockSpec((_NS, _W * 16), lambda: (0, 0)),
            pl.BlockSpec((_BATCH, 1), lambda: (0, 0)),
            pl.BlockSpec((1, 1), lambda: (0, 0), memory_space=pltpu.SMEM),
        ],
        out_specs=[pl.BlockSpec((_BATCH, 1), lambda: (0, 0))] * 4,
        out_shape=[jax.ShapeDtypeStruct((_BATCH, 1), jnp.float32)] * 4,
    )(raw_tc.reshape(_NT, 1), partials.reshape(_NS, _W * 16),
      R.reshape(_BATCH, 1), nobs)

    return (raw.reshape(_BATCH), mu.reshape(_BATCH),
            sigma2.reshape(_BATCH), obj.reshape(_BATCH))


# speedup vs baseline: 1.2140x; 1.2140x over previous
import functools

import jax
import jax.numpy as jnp
from jax import lax
from jax.experimental import pallas as pl
from jax.experimental.pallas import tpu as pltpu
from jax.experimental.pallas import tpu_sc as plsc

_BATCH = 64
_THRESH2 = 3.0
_ALPHA = 2.0
_BETA = 1.0
_OBS_R = 256
_OBS_S = 512

_NS = 24
_NT = _BATCH - _NS
_QPB = 4
_ITEMS = _NS * _QPB
_K = _ITEMS // 32
_RPI = _OBS_R // _QPB
_CR = 32
_NCHI = _RPI // _CR
_UNROLL = 8
_GRP = _OBS_S // 16


def _tc_score(z_ref, r_ref, out_ref):
    x = z_ref[0]
    x0 = x[0]
    x1 = x[1]
    x2 = x[2]
    z2 = x0 * x0 + x1 * x1 + x2 * x2
    b_coef = -0.5 / (r_ref[0, 0, 0] * r_ref[0, 0, 0])
    e = jnp.exp(z2 * b_coef)
    scores = jnp.where(z2 < _THRESH2, e, 0.0)
    out_ref[0] = jnp.sum(scores, axis=0, keepdims=True)


def _sc_score(z_hbm, bc_hbm, out_hbm, b0a, b0b, b1a, b1b, b2a, b2b,
              bc_all, o_v, sem_a, sem_b):
    wid = lax.axis_index("s") * 2 + lax.axis_index("c")
    i0 = wid * _K
    bufs = [(b0a, b1a, b2a), (b0b, b1b, b2b)]
    sems = [sem_a, sem_b]

    pltpu.sync_copy(bc_hbm, bc_all)

    chunks = []
    for j in range(_K):
        item = i0 + j
        b = _NT + item // _QPB
        r0 = (item % _QPB) * _RPI
        for c in range(_NCHI):
            chunks.append((b, r0 + c * _CR))

    def start(ci, d):
        b, rs = chunks[ci]
        return tuple(
            pltpu.async_copy(z_hbm.at[b, k, pl.ds(rs, _CR)],
                             bufs[d][k], sems[d])
            for k in range(3))

    handles = {0: start(0, 0)}
    for j in range(_K):
        item = i0 + j
        b = _NT + item // _QPB
        bcoef = bc_all[b - _NT, :]
        acc = (jnp.zeros((16,), jnp.float32),) * 4
        for c in range(_NCHI):
            ci = j * _NCHI + c
            d = ci % 2
            if ci + 1 < len(chunks):
                handles[ci + 1] = start(ci + 1, (ci + 1) % 2)
            for hd in handles.pop(ci):
                hd.wait()
            b0, b1, b2 = bufs[d]

            def row_step(r, carry):
                def grp_step(g, carry2):
                    accs = list(carry2)
                    base = g * (16 * _UNROLL)
                    for u in range(_UNROLL):
                        sl = pl.ds(base + 16 * u, 16)
                        x0 = b0[r, sl]
                        x1 = b1[r, sl]
                        x2 = b2[r, sl]
                        z2 = x0 * x0 + x1 * x1 + x2 * x2
                        e = jnp.exp(z2 * bcoef)
                        accs[u % 4] = accs[u % 4] + jnp.where(
                            z2 < _THRESH2, e, 0.0)
                    return tuple(accs)

                return lax.fori_loop(0, _GRP // _UNROLL, grp_step, carry)

            acc = lax.fori_loop(0, _CR, row_step, acc)
        o_v[...] = (acc[0] + acc[1]) + (acc[2] + acc[3])
        q = item % _QPB
        pltpu.sync_copy(o_v, out_hbm.at[item // _QPB, pl.ds(q * 16, 16)])


def _finish_body(tcp_ref, scp_ref, r_ref, nobs_ref, raw_ref, mu_ref,
                 s2_ref, obj_ref):
    raw_tc = jnp.sum(tcp_ref[...], axis=2)
    raw_sc = jnp.sum(scp_ref[...], axis=1, keepdims=True)
    raw = jnp.concatenate([raw_tc, raw_sc], axis=0).reshape(1, _BATCH)
    r = r_ref[...]
    a = 1.0 / (r * r)
    b = 0.5 * a
    t2 = _THRESH2
    mu = (1.0 - jnp.exp(-b * t2)) / (4.0 * b)
    mean_s2 = (1.0 - jnp.exp(-2.0 * b * t2)) / (8.0 * b)
    sigma2 = mean_s2 - mu * mu
    n = nobs_ref[0, 0]
    mu = n * mu
    sigma2 = n * sigma2
    raw_ref[...] = raw
    mu_ref[...] = mu
    s2_ref[...] = sigma2
    obj_ref[...] = raw - _ALPHA * mu - _BETA * sigma2


@functools.partial(jax.jit, static_argnames=())
def kernel(z, R, num_obs):
    zt = jnp.transpose(z, (0, 3, 1, 2))
    bcast = jnp.broadcast_to((-0.5 / (R * R))[_NT:, None], (_NS, 16))
    mesh = plsc.VectorSubcoreMesh(core_axis_name="c", subcore_axis_name="s")

    sc_part = pl.kernel(
        _sc_score,
        mesh=mesh,
        compiler_params=pltpu.CompilerParams(needs_layout_passes=False),
        out_type=jax.ShapeDtypeStruct((_NS, _QPB * 16), jnp.float32),
        scratch_types=(
            [pltpu.VMEM((_CR, _OBS_S), jnp.float32)] * 6
            + [pltpu.VMEM((_NS, 16), jnp.float32),
               pltpu.VMEM((16,), jnp.float32)]
            + [pltpu.SemaphoreType.DMA] * 2
        ),
    )(zt, bcast)

    tc_part = pl.pallas_call(
        _tc_score,
        grid=(_NT,),
        in_specs=[
            pl.BlockSpec((1, 3, _OBS_R, _OBS_S), lambda bb: (bb, 0, 0, 0)),
            pl.BlockSpec((1, 1, 1), lambda bb: (bb, 0, 0),
                         memory_space=pltpu.SMEM),
        ],
        out_specs=pl.BlockSpec((1, 1, _OBS_S), lambda bb: (bb, 0, 0)),
        out_shape=jax.ShapeDtypeStruct((_NT, 1, _OBS_S), jnp.float32),
    )(zt, R.reshape(_BATCH, 1, 1))

    nobs = jnp.asarray(num_obs, jnp.float32).reshape(1, 1)
    raw, mu, sigma2, obj = pl.pallas_call(
        _finish_body,
        in_specs=[
            pl.BlockSpec((_NT, 1, _OBS_S), lambda: (0, 0, 0)),
            pl.BlockSpec((_NS, _QPB * 16), lambda: (0, 0)),
            pl.BlockSpec((1, _BATCH), lambda: (0, 0)),
            pl.BlockSpec((1, 1), lambda: (0, 0), memory_space=pltpu.SMEM),
        ],
        out_specs=[pl.BlockSpec((1, _BATCH), lambda: (0, 0))] * 4,
        out_shape=[jax.ShapeDtypeStruct((1, _BATCH), jnp.float32)] * 4,
    )(tc_part, sc_part, R.reshape(1, _BATCH), nobs)

    return (raw.reshape(_BATCH), mu.reshape(_BATCH),
            sigma2.reshape(_BATCH), obj.reshape(_BATCH))


# speedup vs baseline: 1.2987x; 1.0698x over previous
import functools

import jax
import jax.numpy as jnp
from jax import lax
from jax.experimental import pallas as pl
from jax.experimental.pallas import tpu as pltpu
from jax.experimental.pallas import tpu_sc as plsc

_BATCH = 64
_THRESH2 = 3.0
_ALPHA = 2.0
_BETA = 1.0
_OBS_R = 256
_OBS_S = 512

_NS = 32
_NT = _BATCH - _NS
_QPB = 4
_ITEMS = _NS * _QPB
_K = _ITEMS // 32
_RPI = _OBS_R // _QPB
_CR = 32
_NCHI = _RPI // _CR
_UNROLL = 8
_GRP = _OBS_S // 16


def _tc_score(z_ref, r_ref, out_ref):
    x = z_ref[0]
    x0 = x[0]
    x1 = x[1]
    x2 = x[2]
    z2 = x0 * x0 + x1 * x1 + x2 * x2
    b_coef = -0.5 / (r_ref[0, 0, 0] * r_ref[0, 0, 0])
    e = jnp.exp(z2 * b_coef)
    scores = jnp.where(z2 < _THRESH2, e, 0.0)
    out_ref[0] = jnp.sum(scores, axis=0, keepdims=True)


def _sc_score(z_hbm, bc_hbm, out_hbm, b0a, b0b, b1a, b1b, b2a, b2b,
              bc_all, o_v, sem_a, sem_b):
    wid = lax.axis_index("s") * 2 + lax.axis_index("c")
    i0 = wid * _K
    bufs = [(b0a, b1a, b2a), (b0b, b1b, b2b)]
    sems = [sem_a, sem_b]

    pltpu.sync_copy(bc_hbm, bc_all)

    chunks = []
    for j in range(_K):
        item = i0 + j
        b = _NT + item // _QPB
        r0 = (item % _QPB) * _RPI
        for c in range(_NCHI):
            chunks.append((b, r0 + c * _CR))

    def start(ci, d):
        b, rs = chunks[ci]
        return tuple(
            pltpu.async_copy(z_hbm.at[b, k, pl.ds(rs, _CR)],
                             bufs[d][k], sems[d])
            for k in range(3))

    handles = {0: start(0, 0)}
    for j in range(_K):
        item = i0 + j
        b = _NT + item // _QPB
        bcoef = bc_all[b - _NT, :]
        acc = (jnp.zeros((16,), jnp.float32),) * 4
        for c in range(_NCHI):
            ci = j * _NCHI + c
            d = ci % 2
            if ci + 1 < len(chunks):
                handles[ci + 1] = start(ci + 1, (ci + 1) % 2)
            for hd in handles.pop(ci):
                hd.wait()
            b0, b1, b2 = bufs[d]

            def row_step(r, carry):
                def grp_step(g, carry2):
                    accs = list(carry2)
                    base = g * (16 * _UNROLL)
                    for u in range(_UNROLL):
                        sl = pl.ds(base + 16 * u, 16)
                        x0 = b0[r, sl]
                        x1 = b1[r, sl]
                        x2 = b2[r, sl]
                        z2 = x0 * x0 + x1 * x1 + x2 * x2
                        e = jnp.exp(z2 * bcoef)
                        accs[u % 4] = accs[u % 4] + jnp.where(
                            z2 < _THRESH2, e, 0.0)
                    return tuple(accs)

                return lax.fori_loop(0, _GRP // _UNROLL, grp_step, carry)

            acc = lax.fori_loop(0, _CR, row_step, acc)
        o_v[...] = (acc[0] + acc[1]) + (acc[2] + acc[3])
        q = item % _QPB
        pltpu.sync_copy(o_v, out_hbm.at[item // _QPB, pl.ds(q * 16, 16)])


def _finish_body(tcp_ref, scp_ref, r_ref, nobs_ref, raw_ref, mu_ref,
                 s2_ref, obj_ref):
    raw_tc = jnp.sum(tcp_ref[...], axis=2)
    raw_sc = jnp.sum(scp_ref[...], axis=1, keepdims=True)
    raw = jnp.concatenate([raw_tc, raw_sc], axis=0).reshape(1, _BATCH)
    r = r_ref[...]
    a = 1.0 / (r * r)
    b = 0.5 * a
    t2 = _THRESH2
    mu = (1.0 - jnp.exp(-b * t2)) / (4.0 * b)
    mean_s2 = (1.0 - jnp.exp(-2.0 * b * t2)) / (8.0 * b)
    sigma2 = mean_s2 - mu * mu
    n = nobs_ref[0, 0]
    mu = n * mu
    sigma2 = n * sigma2
    raw_ref[...] = raw
    mu_ref[...] = mu
    s2_ref[...] = sigma2
    obj_ref[...] = raw - _ALPHA * mu - _BETA * sigma2


@functools.partial(jax.jit, static_argnames=())
def kernel(z, R, num_obs):
    zt = jnp.transpose(z, (0, 3, 1, 2))
    bcast = jnp.broadcast_to((-0.5 / (R * R))[_NT:, None], (_NS, 16))
    mesh = plsc.VectorSubcoreMesh(core_axis_name="c", subcore_axis_name="s")

    sc_part = pl.kernel(
        _sc_score,
        mesh=mesh,
        compiler_params=pltpu.CompilerParams(needs_layout_passes=False),
        out_type=jax.ShapeDtypeStruct((_NS, _QPB * 16), jnp.float32),
        scratch_types=(
            [pltpu.VMEM((_CR, _OBS_S), jnp.float32)] * 6
            + [pltpu.VMEM((_NS, 16), jnp.float32),
               pltpu.VMEM((16,), jnp.float32)]
            + [pltpu.SemaphoreType.DMA] * 2
        ),
    )(zt, bcast)

    tc_part = pl.pallas_call(
        _tc_score,
        grid=(_NT,),
        in_specs=[
            pl.BlockSpec((1, 3, _OBS_R, _OBS_S), lambda bb: (bb, 0, 0, 0)),
            pl.BlockSpec((1, 1, 1), lambda bb: (bb, 0, 0),
                         memory_space=pltpu.SMEM),
        ],
        out_specs=pl.BlockSpec((1, 1, _OBS_S), lambda bb: (bb, 0, 0)),
        out_shape=jax.ShapeDtypeStruct((_NT, 1, _OBS_S), jnp.float32),
    )(zt, R.reshape(_BATCH, 1, 1))

    nobs = jnp.asarray(num_obs, jnp.float32).reshape(1, 1)
    raw, mu, sigma2, obj = pl.pallas_call(
        _finish_body,
        in_specs=[
            pl.BlockSpec((_NT, 1, _OBS_S), lambda: (0, 0, 0)),
            pl.BlockSpec((_NS, _QPB * 16), lambda: (0, 0)),
            pl.BlockSpec((1, _BATCH), lambda: (0, 0)),
            pl.BlockSpec((1, 1), lambda: (0, 0), memory_space=pltpu.SMEM),
        ],
        out_specs=[pl.BlockSpec((1, _BATCH), lambda: (0, 0))] * 4,
        out_shape=[jax.ShapeDtypeStruct((1, _BATCH), jnp.float32)] * 4,
    )(tc_part, sc_part, R.reshape(1, _BATCH), nobs)

    return (raw.reshape(_BATCH), mu.reshape(_BATCH),
            sigma2.reshape(_BATCH), obj.reshape(_BATCH))


# speedup vs baseline: 1.3117x; 1.0100x over previous
import functools

import jax
import jax.numpy as jnp
from jax import lax
from jax.experimental import pallas as pl
from jax.experimental.pallas import tpu as pltpu
from jax.experimental.pallas import tpu_sc as plsc

_BATCH = 64
_THRESH2 = 3.0
_ALPHA = 2.0
_BETA = 1.0
_OBS_R = 256
_OBS_S = 512

_NS = 32
_NT = _BATCH - _NS
_QPB = 4
_ITEMS = _NS * _QPB
_K = _ITEMS // 32
_RPI = _OBS_R // _QPB
_CR = 32
_NCHI = _RPI // _CR
_UNROLL = 16
_NACC = 8
_GRP = _OBS_S // 16


def _tc_score(z_ref, r_ref, out_ref):
    x = z_ref[0]
    x0 = x[0]
    x1 = x[1]
    x2 = x[2]
    z2 = x0 * x0 + x1 * x1 + x2 * x2
    b_coef = -0.5 / (r_ref[0, 0, 0] * r_ref[0, 0, 0])
    e = jnp.exp(z2 * b_coef)
    scores = jnp.where(z2 < _THRESH2, e, 0.0)
    out_ref[0] = jnp.sum(scores, axis=0, keepdims=True)


def _sc_score(z_hbm, bc_hbm, out_hbm, b0a, b0b, b1a, b1b, b2a, b2b,
              bc_all, o_v, sem_a, sem_b):
    wid = lax.axis_index("s") * 2 + lax.axis_index("c")
    i0 = wid * _K
    bufs = [(b0a, b1a, b2a), (b0b, b1b, b2b)]
    sems = [sem_a, sem_b]

    pltpu.sync_copy(bc_hbm, bc_all)

    chunks = []
    for j in range(_K):
        item = i0 + j
        b = _NT + item // _QPB
        r0 = (item % _QPB) * _RPI
        for c in range(_NCHI):
            chunks.append((b, r0 + c * _CR))

    def start(ci, d):
        b, rs = chunks[ci]
        return tuple(
            pltpu.async_copy(z_hbm.at[b, k, pl.ds(rs, _CR)],
                             bufs[d][k], sems[d])
            for k in range(3))

    handles = {0: start(0, 0)}
    for j in range(_K):
        item = i0 + j
        b = _NT + item // _QPB
        bcoef = bc_all[b - _NT, :]
        acc = (jnp.zeros((16,), jnp.float32),) * _NACC
        for c in range(_NCHI):
            ci = j * _NCHI + c
            d = ci % 2
            if ci + 1 < len(chunks):
                handles[ci + 1] = start(ci + 1, (ci + 1) % 2)
            for hd in handles.pop(ci):
                hd.wait()
            b0, b1, b2 = bufs[d]

            def row_step(r, carry):
                def grp_step(g, carry2):
                    accs = list(carry2)
                    base = g * (16 * _UNROLL)
                    for u in range(_UNROLL):
                        sl = pl.ds(base + 16 * u, 16)
                        x0 = b0[r, sl]
                        x1 = b1[r, sl]
                        x2 = b2[r, sl]
                        z2 = x0 * x0 + x1 * x1 + x2 * x2
                        e = jnp.exp(z2 * bcoef)
                        accs[u % _NACC] = accs[u % _NACC] + jnp.where(
                            z2 < _THRESH2, e, 0.0)
                    return tuple(accs)

                return lax.fori_loop(0, _GRP // _UNROLL, grp_step, carry)

            acc = lax.fori_loop(0, _CR, row_step, acc)
        t01 = (acc[0] + acc[1]) + (acc[2] + acc[3])
        t23 = (acc[4] + acc[5]) + (acc[6] + acc[7])
        o_v[...] = t01 + t23
        q = item % _QPB
        pltpu.sync_copy(o_v, out_hbm.at[item // _QPB, pl.ds(q * 16, 16)])


def _finish_body(tcp_ref, scp_ref, r_ref, nobs_ref, raw_ref, mu_ref,
                 s2_ref, obj_ref):
    raw_tc = jnp.sum(tcp_ref[...], axis=2)
    raw_sc = jnp.sum(scp_ref[...], axis=1, keepdims=True)
    raw = jnp.concatenate([raw_tc, raw_sc], axis=0).reshape(1, _BATCH)
    r = r_ref[...]
    a = 1.0 / (r * r)
    b = 0.5 * a
    t2 = _THRESH2
    mu = (1.0 - jnp.exp(-b * t2)) / (4.0 * b)
    mean_s2 = (1.0 - jnp.exp(-2.0 * b * t2)) / (8.0 * b)
    sigma2 = mean_s2 - mu * mu
    n = nobs_ref[0, 0]
    mu = n * mu
    sigma2 = n * sigma2
    raw_ref[...] = raw
    mu_ref[...] = mu
    s2_ref[...] = sigma2
    obj_ref[...] = raw - _ALPHA * mu - _BETA * sigma2


@functools.partial(jax.jit, static_argnames=())
def kernel(z, R, num_obs):
    zt = jnp.transpose(z, (0, 3, 1, 2))
    bcast = jnp.broadcast_to((-0.5 / (R * R))[_NT:, None], (_NS, 16))
    mesh = plsc.VectorSubcoreMesh(core_axis_name="c", subcore_axis_name="s")

    sc_part = pl.kernel(
        _sc_score,
        mesh=mesh,
        compiler_params=pltpu.CompilerParams(needs_layout_passes=False),
        out_type=jax.ShapeDtypeStruct((_NS, _QPB * 16), jnp.float32),
        scratch_types=(
            [pltpu.VMEM((_CR, _OBS_S), jnp.float32)] * 6
            + [pltpu.VMEM((_NS, 16), jnp.float32),
               pltpu.VMEM((16,), jnp.float32)]
            + [pltpu.SemaphoreType.DMA] * 2
        ),
    )(zt, bcast)

    tc_part = pl.pallas_call(
        _tc_score,
        grid=(_NT,),
        in_specs=[
            pl.BlockSpec((1, 3, _OBS_R, _OBS_S), lambda bb: (bb, 0, 0, 0)),
            pl.BlockSpec((1, 1, 1), lambda bb: (bb, 0, 0),
                         memory_space=pltpu.SMEM),
        ],
        out_specs=pl.BlockSpec((1, 1, _OBS_S), lambda bb: (bb, 0, 0)),
        out_shape=jax.ShapeDtypeStruct((_NT, 1, _OBS_S), jnp.float32),
    )(zt, R.reshape(_BATCH, 1, 1))

    nobs = jnp.asarray(num_obs, jnp.float32).reshape(1, 1)
    raw, mu, sigma2, obj = pl.pallas_call(
        _finish_body,
        in_specs=[
            pl.BlockSpec((_NT, 1, _OBS_S), lambda: (0, 0, 0)),
            pl.BlockSpec((_NS, _QPB * 16), lambda: (0, 0)),
            pl.BlockSpec((1, _BATCH), lambda: (0, 0)),
            pl.BlockSpec((1, 1), lambda: (0, 0), memory_space=pltpu.SMEM),
        ],
        out_specs=[pl.BlockSpec((1, _BATCH), lambda: (0, 0))] * 4,
        out_shape=[jax.ShapeDtypeStruct((1, _BATCH), jnp.float32)] * 4,
    )(tc_part, sc_part, R.reshape(1, _BATCH), nobs)

    return (raw.reshape(_BATCH), mu.reshape(_BATCH),
            sigma2.reshape(_BATCH), obj.reshape(_BATCH))
